# baseline (speedup 1.0000x reference)
import functools

import jax
import jax.numpy as jnp
from jax import lax
from jax.experimental import pallas as pl
from jax.experimental.pallas import tpu as pltpu
from jax.experimental.pallas import tpu_sc as plsc

_NC = 2
_NS = 16
_NW = _NC * _NS
_L = 16

_C = 64
_REC = 3 * _C


def _pack_edges(dst, src, values, nchunks):
    E = values.shape[0]
    epw = E // _NW
    pad = nchunks * _C - epw

    def tile_pad(x):
        return jnp.pad(x.reshape(_NW, epw), ((0, 0), (0, pad)))

    s = tile_pad(src).reshape(_NW, nchunks, 1, _C)
    d = tile_pad(dst).reshape(_NW, nchunks, 1, _C)
    v = tile_pad(jax.lax.bitcast_convert_type(values, jnp.int32))
    v = v.reshape(_NW, nchunks, 1, _C)
    return jnp.concatenate([s, d, v], axis=2).reshape(-1)


def _sc_spmm_partials(dst, src, values, b):
    E = values.shape[0]
    N, D = b.shape
    EPW = E // _NW
    NF = 160
    RT = 624
    ZR = 48
    TAIL = N - RT * _NS
    assert EPW * _NW == E and NF * _C >= EPW and NF % 4 == 0
    assert RT % ZR == 0 and TAIL % 8 == 0 and TAIL <= ZR and D % _L == 0

    packed = _pack_edges(dst, src, values, NF)

    mesh = plsc.VectorSubcoreMesh(core_axis_name="c", subcore_axis_name="s")

    scratch = dict(
        acc_sh=pltpu.VMEM_SHARED((N, D), jnp.float32),
        zero_v=pltpu.VMEM((ZR, D), jnp.float32),
    )
    for k in range(4):
        scratch[f"rows{k}"] = pltpu.VMEM((_C, D), jnp.float32)
        scratch[f"pk{k}"] = pltpu.VMEM((_REC,), jnp.int32)
        scratch[f"db{k}"] = pltpu.VMEM((_C,), jnp.int32)
        scratch[f"gsem{k}"] = pltpu.SemaphoreType.DMA
        scratch[f"ssem{k}"] = pltpu.SemaphoreType.DMA
        scratch[f"csem{k}"] = pltpu.SemaphoreType.DMA

    @functools.partial(
        pl.kernel,
        mesh=mesh,
        out_type=jax.ShapeDtypeStruct((_NC, N, D), jnp.float32),
        scratch_types=scratch,
    )
    def spmm(pk_hbm, b_hbm, out_hbm, **scr):
        acc_sh = scr["acc_sh"]
        zero_v = scr["zero_v"]
        rows = [scr[f"rows{k}"] for k in range(4)]
        pk = [scr[f"pk{k}"] for k in range(4)]
        db = [scr[f"db{k}"] for k in range(4)]
        gsem = [scr[f"gsem{k}"] for k in range(4)]
        ssem = [scr[f"ssem{k}"] for k in range(4)]
        csem = [scr[f"csem{k}"] for k in range(4)]

        cid = lax.axis_index("c")
        sid = lax.axis_index("s")
        w = sid * _NC + cid
        pbase = w * (NF * _REC)

        zvec = jnp.zeros((_L,), jnp.float32)

        def zrow(r, _):
            for j in range(D // _L):
                zero_v[r, pl.ds(j * _L, _L)] = zvec
            return 0

        lax.fori_loop(0, ZR, zrow, 0)
        zcps = []
        for z in range(RT // ZR):
            off = pl.multiple_of(sid * RT + z * ZR, 8)
            zcps.append(pltpu.make_async_copy(
                zero_v, acc_sh.at[pl.ds(off, ZR)], csem[0]))
        zcps.append(pltpu.make_async_copy(
            zero_v.at[pl.ds(0, TAIL)],
            acc_sh.at[pl.ds(N - TAIL, TAIL)], csem[0]))
        for cp in zcps:
            cp.start()
        for cp in zcps:
            cp.wait()

        plsc.subcore_barrier()

        def pkd(i, k):
            return pltpu.make_async_copy(
                pk_hbm.at[pl.ds(pbase + i * _REC, _REC)], pk[k], ssem[k])

        def gat(k):
            return pltpu.make_async_copy(
                b_hbm.at[pk[k].at[pl.ds(0, _C)]], rows[k], gsem[k])

        def scat(k):
            return pltpu.async_copy(rows[k], acc_sh.at[db[k]], csem[k],
                                    add=True)

        def scat_wait(k):
            pltpu.make_async_copy(rows[k], acc_sh.at[db[k]], csem[k]).wait()

        dnums = lax.GatherDimensionNumbers(
            offset_dims=(), collapsed_slice_dims=(0,), start_index_map=(0,))

        def scale(k):
            buf = rows[k]

            def group(g, _):
                v_i32 = pk[k][pl.ds(2 * _C + g * _L, _L)]
                vals16 = lax.bitcast_convert_type(v_i32, jnp.float32)
                for e in range(_L):
                    lane = jnp.full((_L, 1), e, jnp.int32)
                    v16 = lax.gather(
                        vals16, lane, dnums, (1,),
                        mode=lax.GatherScatterMode.PROMISE_IN_BOUNDS)
                    for j in range(D // _L):
                        sl = pl.ds(j * _L, _L)
                        buf[g * _L + e, sl] = buf[g * _L + e, sl] * v16
                return 0

            lax.fori_loop(0, _C // _L, group, 0)

        def phase(i, k, fire_pk=True, fire_gat=True, wait_scat=True):
            pltpu.make_async_copy(
                b_hbm.at[pk[k].at[pl.ds(0, _C)]], rows[k], gsem[k]).wait()
            if fire_pk:
                pkd(i + 3, (k + 3) % 4).start()
            if wait_scat:
                scat_wait((k + 2) % 4)
            for g in range(_C // _L):
                db[k][pl.ds(g * _L, _L)] = pk[k][pl.ds(_C + g * _L, _L)]
            scale(k)
            scat(k)
            if fire_gat:
                pkd(i + 2, (k + 2) % 4).wait()
                gat((k + 2) % 4).start()

        pkd(0, 0).start()
        pkd(1, 1).start()
        pkd(2, 2).start()
        pkd(0, 0).wait()
        gat(0).start()
        pkd(1, 1).wait()
        gat(1).start()

        phase(0, 0, wait_scat=False)
        phase(1, 1, wait_scat=False)
        phase(2, 2)
        phase(3, 3)

        def quad(g, _):
            i0 = 4 * g + 4
            phase(i0, 0)
            phase(i0 + 1, 1)
            phase(i0 + 2, 2)
            phase(i0 + 3, 3)
            return 0

        lax.fori_loop(0, NF // 4 - 2, quad, 0)

        phase(NF - 4, 0)
        phase(NF - 3, 1, fire_pk=False)
        phase(NF - 2, 2, fire_pk=False, fire_gat=False)
        phase(NF - 1, 3, fire_pk=False, fire_gat=False)
        scat_wait(2)
        scat_wait(3)

        plsc.subcore_barrier()

        off = pl.multiple_of(sid * RT, 8)
        out1 = pltpu.make_async_copy(acc_sh.at[pl.ds(off, RT)],
                                     out_hbm.at[cid, pl.ds(off, RT)],
                                     gsem[0])
        out2 = pltpu.make_async_copy(acc_sh.at[pl.ds(N - TAIL, TAIL)],
                                     out_hbm.at[cid, pl.ds(N - TAIL, TAIL)],
                                     gsem[1])
        out1.start()
        out2.start()
        out1.wait()
        out2.wait()

    return spmm(packed, b)


def _merge_body(p_ref, o_ref):
    o_ref[...] = p_ref[0] + p_ref[1]


def kernel(indices, values, shape, b):
    del shape
    N, D = b.shape
    partials = _sc_spmm_partials(indices[0], indices[1], values, b)
    nblk = 10
    rb = N // nblk
    return pl.pallas_call(
        _merge_body,
        grid=(nblk,),
        in_specs=[pl.BlockSpec((_NC, rb, D), lambda i: (0, i, 0))],
        out_specs=pl.BlockSpec((rb, D), lambda i: (i, 0)),
        out_shape=jax.ShapeDtypeStruct((N, D), jnp.float32),
    )(partials)


# speedup vs baseline: 1.0003x; 1.0003x over previous
import functools

import jax
import jax.numpy as jnp
from jax import lax
from jax.experimental import pallas as pl
from jax.experimental.pallas import tpu as pltpu
from jax.experimental.pallas import tpu_sc as plsc

_NC = 2
_NS = 16
_NW = _NC * _NS
_L = 16

_C = 64
_REC = 3 * _C


def _pack_edges(dst, src, values, nchunks):
    E = values.shape[0]
    epw = E // _NW
    pad = nchunks * _C - epw

    def tile_pad(x):
        return jnp.pad(x.reshape(_NW, epw), ((0, 0), (0, pad)))

    s = tile_pad(src).reshape(_NW, nchunks, 1, _C)
    d = tile_pad(dst).reshape(_NW, nchunks, 1, _C)
    v = tile_pad(jax.lax.bitcast_convert_type(values, jnp.int32))
    v = v.reshape(_NW, nchunks, 1, _C)
    return jnp.concatenate([s, d, v], axis=2).reshape(-1)


def _sc_spmm_partials(dst, src, values, b):
    E = values.shape[0]
    N, D = b.shape
    EPW = E // _NW
    NF = 160
    RT = 624
    ZR = 48
    TAIL = N - RT * _NS
    assert EPW * _NW == E and NF * _C >= EPW and NF % 4 == 0
    assert RT % ZR == 0 and TAIL % 8 == 0 and TAIL <= ZR and D % _L == 0

    packed = _pack_edges(dst, src, values, NF)

    mesh = plsc.VectorSubcoreMesh(core_axis_name="c", subcore_axis_name="s")

    scratch = dict(
        acc_sh=pltpu.VMEM_SHARED((N, D), jnp.float32),
        zero_v=pltpu.VMEM((ZR, D), jnp.float32),
    )
    for k in range(4):
        scratch[f"rows{k}"] = pltpu.VMEM((_C, D), jnp.float32)
        scratch[f"pk{k}"] = pltpu.VMEM((_REC,), jnp.int32)
        scratch[f"sb{k}"] = pltpu.VMEM((_C,), jnp.int32)
        scratch[f"db{k}"] = pltpu.VMEM((_C,), jnp.int32)
        scratch[f"gsem{k}"] = pltpu.SemaphoreType.DMA
        scratch[f"ssem{k}"] = pltpu.SemaphoreType.DMA
        scratch[f"csem{k}"] = pltpu.SemaphoreType.DMA

    @functools.partial(
        pl.kernel,
        mesh=mesh,
        out_type=jax.ShapeDtypeStruct((_NC, N, D), jnp.float32),
        scratch_types=scratch,
    )
    def spmm(pk_hbm, b_hbm, out_hbm, **scr):
        acc_sh = scr["acc_sh"]
        zero_v = scr["zero_v"]
        rows = [scr[f"rows{k}"] for k in range(4)]
        pk = [scr[f"pk{k}"] for k in range(4)]
        sb = [scr[f"sb{k}"] for k in range(4)]
        db = [scr[f"db{k}"] for k in range(4)]
        gsem = [scr[f"gsem{k}"] for k in range(4)]
        ssem = [scr[f"ssem{k}"] for k in range(4)]
        csem = [scr[f"csem{k}"] for k in range(4)]

        cid = lax.axis_index("c")
        sid = lax.axis_index("s")
        w = sid * _NC + cid
        pbase = w * (NF * _REC)

        zvec = jnp.zeros((_L,), jnp.float32)

        def zrow(r, _):
            for j in range(D // _L):
                zero_v[r, pl.ds(j * _L, _L)] = zvec
            return 0

        lax.fori_loop(0, ZR, zrow, 0)
        zcps = []
        for z in range(RT // ZR):
            off = pl.multiple_of(sid * RT + z * ZR, 8)
            zcps.append(pltpu.make_async_copy(
                zero_v, acc_sh.at[pl.ds(off, ZR)], csem[0]))
        zcps.append(pltpu.make_async_copy(
            zero_v.at[pl.ds(0, TAIL)],
            acc_sh.at[pl.ds(N - TAIL, TAIL)], csem[0]))
        for cp in zcps:
            cp.start()
        for cp in zcps:
            cp.wait()

        plsc.subcore_barrier()

        def pkd(i, k):
            return pltpu.make_async_copy(
                pk_hbm.at[pl.ds(pbase + i * _REC, _REC)], pk[k], ssem[k])

        def stage_src(k):
            for g in range(_C // _L):
                sb[k][pl.ds(g * _L, _L)] = pk[k][pl.ds(g * _L, _L)]

        def gat(k):
            return pltpu.make_async_copy(b_hbm.at[sb[k]], rows[k], gsem[k])

        def scat(k):
            return pltpu.async_copy(rows[k], acc_sh.at[db[k]], csem[k],
                                    add=True)

        def scat_wait(k):
            pltpu.make_async_copy(rows[k], acc_sh.at[db[k]], csem[k]).wait()

        dnums = lax.GatherDimensionNumbers(
            offset_dims=(), collapsed_slice_dims=(0,), start_index_map=(0,))

        def scale(k):
            buf = rows[k]

            def group(g, _):
                v_i32 = pk[k][pl.ds(2 * _C + g * _L, _L)]
                vals16 = lax.bitcast_convert_type(v_i32, jnp.float32)
                for e in range(_L):
                    lane = jnp.full((_L, 1), e, jnp.int32)
                    v16 = lax.gather(
                        vals16, lane, dnums, (1,),
                        mode=lax.GatherScatterMode.PROMISE_IN_BOUNDS)
                    for j in range(D // _L):
                        sl = pl.ds(j * _L, _L)
                        buf[g * _L + e, sl] = buf[g * _L + e, sl] * v16
                return 0

            lax.fori_loop(0, _C // _L, group, 0)

        def phase(i, k, fire_pk=True, fire_gat=True, wait_scat=True):
            pltpu.make_async_copy(
                b_hbm.at[sb[k]], rows[k], gsem[k]).wait()
            if fire_pk:
                pkd(i + 3, (k + 3) % 4).start()
            if wait_scat:
                scat_wait((k + 2) % 4)
            for g in range(_C // _L):
                db[k][pl.ds(g * _L, _L)] = pk[k][pl.ds(_C + g * _L, _L)]
            scale(k)
            scat(k)
            if fire_gat:
                pkd(i + 2, (k + 2) % 4).wait()
                stage_src((k + 2) % 4)
                gat((k + 2) % 4).start()

        pkd(0, 0).start()
        pkd(1, 1).start()
        pkd(2, 2).start()
        pkd(0, 0).wait()
        stage_src(0)
        gat(0).start()
        pkd(1, 1).wait()
        stage_src(1)
        gat(1).start()

        phase(0, 0, wait_scat=False)
        phase(1, 1, wait_scat=False)
        phase(2, 2)
        phase(3, 3)

        def quad(g, _):
            i0 = 4 * g + 4
            phase(i0, 0)
            phase(i0 + 1, 1)
            phase(i0 + 2, 2)
            phase(i0 + 3, 3)
            return 0

        lax.fori_loop(0, NF // 4 - 2, quad, 0)

        phase(NF - 4, 0)
        phase(NF - 3, 1, fire_pk=False)
        phase(NF - 2, 2, fire_pk=False, fire_gat=False)
        phase(NF - 1, 3, fire_pk=False, fire_gat=False)
        scat_wait(2)
        scat_wait(3)

        plsc.subcore_barrier()

        off = pl.multiple_of(sid * RT, 8)
        out1 = pltpu.make_async_copy(acc_sh.at[pl.ds(off, RT)],
                                     out_hbm.at[cid, pl.ds(off, RT)],
                                     gsem[0])
        out2 = pltpu.make_async_copy(acc_sh.at[pl.ds(N - TAIL, TAIL)],
                                     out_hbm.at[cid, pl.ds(N - TAIL, TAIL)],
                                     gsem[1])
        out1.start()
        out2.start()
        out1.wait()
        out2.wait()

    return spmm(packed, b)


def _merge_body(p_ref, o_ref):
    o_ref[...] = p_ref[0] + p_ref[1]


def kernel(indices, values, shape, b):
    del shape
    N, D = b.shape
    partials = _sc_spmm_partials(indices[0], indices[1], values, b)
    nblk = 10
    rb = N // nblk
    return pl.pallas_call(
        _merge_body,
        grid=(nblk,),
        in_specs=[pl.BlockSpec((_NC, rb, D), lambda i: (0, i, 0))],
        out_specs=pl.BlockSpec((rb, D), lambda i: (i, 0)),
        out_shape=jax.ShapeDtypeStruct((N, D), jnp.float32),
    )(partials)


# speedup vs baseline: 2.9781x; 2.9772x over previous
import functools

import jax
import jax.numpy as jnp
from jax import lax
from jax.experimental import pallas as pl
from jax.experimental.pallas import tpu as pltpu
from jax.experimental.pallas import tpu_sc as plsc

_NC = 2
_NS = 16
_NW = _NC * _NS
_L = 16


def _sc_spmm_partials(dst, src, values, b):
    E = values.shape[0]
    N, D = b.shape
    EPW = E // _NW
    C = 64
    NF = EPW // C
    REM = NF * C
    CR = EPW - REM
    RT = 624
    ZR = 48
    TAIL = N - RT * _NS
    assert EPW * _NW == E and NF % 4 == 0 and NF >= 12 and CR in (0, _L)
    assert RT % ZR == 0 and TAIL % 8 == 0 and TAIL <= ZR and D % _L == 0

    mesh = plsc.VectorSubcoreMesh(core_axis_name="c", subcore_axis_name="s")

    scratch = dict(
        acc_sh=pltpu.VMEM_SHARED((N, D), jnp.float32),
        vals_v=pltpu.VMEM((EPW,), jnp.float32),
        zero_v=pltpu.VMEM((ZR, D), jnp.float32),
        rems_v=pltpu.VMEM((_L,), jnp.int32),
        remd_v=pltpu.VMEM((_L,), jnp.int32),
    )
    for k in range(4):
        scratch[f"rows{k}"] = pltpu.VMEM((C, D), jnp.float32)
        scratch[f"sb{k}"] = pltpu.VMEM((C,), jnp.int32)
        scratch[f"db{k}"] = pltpu.VMEM((C,), jnp.int32)
        scratch[f"gsem{k}"] = pltpu.SemaphoreType.DMA
        scratch[f"ssem{k}"] = pltpu.SemaphoreType.DMA
        scratch[f"dsem{k}"] = pltpu.SemaphoreType.DMA
        scratch[f"csem{k}"] = pltpu.SemaphoreType.DMA

    @functools.partial(
        pl.kernel,
        mesh=mesh,
        out_type=jax.ShapeDtypeStruct((_NC, N, D), jnp.float32),
        scratch_types=scratch,
    )
    def spmm(dst_hbm, src_hbm, vals_hbm, b_hbm, out_hbm, **scr):
        acc_sh = scr["acc_sh"]
        vals_v = scr["vals_v"]
        zero_v = scr["zero_v"]
        rems_v = scr["rems_v"]
        remd_v = scr["remd_v"]
        rows = [scr[f"rows{k}"] for k in range(4)]
        sb = [scr[f"sb{k}"] for k in range(4)]
        db = [scr[f"db{k}"] for k in range(4)]
        gsem = [scr[f"gsem{k}"] for k in range(4)]
        ssem = [scr[f"ssem{k}"] for k in range(4)]
        dsem = [scr[f"dsem{k}"] for k in range(4)]
        csem = [scr[f"csem{k}"] for k in range(4)]

        cid = lax.axis_index("c")
        sid = lax.axis_index("s")
        w = sid * _NC + cid
        ebase = w * EPW

        zvec = jnp.zeros((_L,), jnp.float32)

        def zrow(r, _):
            for j in range(D // _L):
                zero_v[r, pl.ds(j * _L, _L)] = zvec
            return 0

        lax.fori_loop(0, ZR, zrow, 0)
        pltpu.make_async_copy(vals_hbm.at[pl.ds(ebase, EPW)], vals_v,
                              gsem[0]).start()
        zcps = []
        for z in range(RT // ZR):
            off = pl.multiple_of(sid * RT + z * ZR, 8)
            zcps.append(pltpu.make_async_copy(
                zero_v, acc_sh.at[pl.ds(off, ZR)], csem[0]))
        zcps.append(pltpu.make_async_copy(
            zero_v.at[pl.ds(0, TAIL)],
            acc_sh.at[pl.ds(N - TAIL, TAIL)], csem[0]))
        for cp in zcps:
            cp.start()
        for cp in zcps:
            cp.wait()
        pltpu.make_async_copy(vals_hbm.at[pl.ds(ebase, EPW)], vals_v,
                              gsem[0]).wait()

        plsc.subcore_barrier()

        def sidx(i, k):
            return pltpu.make_async_copy(
                src_hbm.at[pl.ds(ebase + i * C, C)], sb[k], ssem[k])

        def didx(i, k):
            return pltpu.make_async_copy(
                dst_hbm.at[pl.ds(ebase + i * C, C)], db[k], dsem[k])

        def gat(k):
            return pltpu.make_async_copy(b_hbm.at[sb[k]], rows[k], gsem[k])

        def scat(k):
            return pltpu.async_copy(rows[k], acc_sh.at[db[k]], csem[k],
                                    add=True)

        def scat_wait(k):
            pltpu.make_async_copy(rows[k], acc_sh.at[db[k]], csem[k]).wait()

        dnums = lax.GatherDimensionNumbers(
            offset_dims=(), collapsed_slice_dims=(0,), start_index_map=(0,))

        def scale(buf, cb, ngroups):
            def group(g, _):
                vals16 = vals_v[pl.ds(cb + g * _L, _L)]
                for e in range(_L):
                    lane = jnp.full((_L, 1), e, jnp.int32)
                    v16 = lax.gather(
                        vals16, lane, dnums, (1,),
                        mode=lax.GatherScatterMode.PROMISE_IN_BOUNDS)
                    for j in range(D // _L):
                        sl = pl.ds(j * _L, _L)
                        buf[g * _L + e, sl] = buf[g * _L + e, sl] * v16
                return 0

            lax.fori_loop(0, ngroups, group, 0)

        def phase(i, k, fire_sidx=True, fire_didx=True, fire_gat=True,
                  wait_scat=True):
            pltpu.make_async_copy(b_hbm.at[sb[k]], rows[k], gsem[k]).wait()
            if fire_sidx:
                sidx(i + 3, (k + 3) % 4).start()
            if wait_scat:
                scat_wait((k + 2) % 4)
            if fire_didx:
                didx(i + 2, (k + 2) % 4).start()
            scale(rows[k], i * C, C // _L)
            didx(i, k).wait()
            scat(k)
            if fire_gat:
                sidx(i + 2, (k + 2) % 4).wait()
                gat((k + 2) % 4).start()

        sidx(0, 0).start()
        sidx(1, 1).start()
        sidx(2, 2).start()
        didx(0, 0).start()
        didx(1, 1).start()
        sidx(0, 0).wait()
        gat(0).start()
        sidx(1, 1).wait()
        gat(1).start()

        phase(0, 0, wait_scat=False)
        phase(1, 1, wait_scat=False)
        phase(2, 2)
        phase(3, 3)

        def quad(g, _):
            i0 = 4 * g + 4
            phase(i0, 0)
            phase(i0 + 1, 1)
            phase(i0 + 2, 2)
            phase(i0 + 3, 3)
            return 0

        lax.fori_loop(0, NF // 4 - 2, quad, 0)

        phase(NF - 4, 0, fire_sidx=True, fire_didx=True, fire_gat=True)
        phase(NF - 3, 1, fire_sidx=False, fire_didx=True, fire_gat=True)
        phase(NF - 2, 2, fire_sidx=False, fire_didx=False, fire_gat=False,
              wait_scat=False)
        if CR:
            pltpu.make_async_copy(
                src_hbm.at[pl.ds(ebase + REM, CR)], rems_v, ssem[0]).start()
            pltpu.make_async_copy(
                dst_hbm.at[pl.ds(ebase + REM, CR)], remd_v, dsem[0]).start()
        phase(NF - 1, 3, fire_sidx=False, fire_didx=False, fire_gat=False,
              wait_scat=False)
        if CR:
            scat_wait(0)
            pltpu.make_async_copy(
                src_hbm.at[pl.ds(ebase + REM, CR)], rems_v, ssem[0]).wait()
            pltpu.make_async_copy(
                b_hbm.at[rems_v], rows[0].at[pl.ds(0, CR)], gsem[0]).start()
            pltpu.make_async_copy(
                b_hbm.at[rems_v], rows[0].at[pl.ds(0, CR)], gsem[0]).wait()
            scale(rows[0], REM, CR // _L)
            pltpu.make_async_copy(
                dst_hbm.at[pl.ds(ebase + REM, CR)], remd_v, dsem[0]).wait()
            pltpu.sync_copy(rows[0].at[pl.ds(0, CR)], acc_sh.at[remd_v],
                            add=True)
        for k in range(4) if not CR else range(1, 4):
            scat_wait(k)

        plsc.subcore_barrier()

        off = pl.multiple_of(sid * RT, 8)
        out1 = pltpu.make_async_copy(acc_sh.at[pl.ds(off, RT)],
                                     out_hbm.at[cid, pl.ds(off, RT)],
                                     gsem[0])
        out2 = pltpu.make_async_copy(acc_sh.at[pl.ds(N - TAIL, TAIL)],
                                     out_hbm.at[cid, pl.ds(N - TAIL, TAIL)],
                                     gsem[1])
        out1.start()
        out2.start()
        out1.wait()
        out2.wait()

    return spmm(dst, src, values, b)


def _merge_body(p_ref, o_ref):
    o_ref[...] = p_ref[0] + p_ref[1]


def kernel(indices, values, shape, b):
    del shape
    N, D = b.shape
    partials = _sc_spmm_partials(indices[0], indices[1], values, b)
    nblk = 10
    rb = N // nblk
    return pl.pallas_call(
        _merge_body,
        grid=(nblk,),
        in_specs=[pl.BlockSpec((_NC, rb, D), lambda i: (0, i, 0))],
        out_specs=pl.BlockSpec((rb, D), lambda i: (i, 0)),
        out_shape=jax.ShapeDtypeStruct((N, D), jnp.float32),
    )(partials)
